# baseline (speedup 1.0000x reference)
import functools

import jax
import jax.numpy as jnp
from jax import lax
from jax.experimental import pallas as pl
from jax.experimental.pallas import tpu as pltpu
from jax.experimental.pallas import tpu_sc as plsc

VOCAB = 1000000
EMBED_DIM = 64
N_KNOWN = 16384
N_OOV = 4096

NC = 2
NS = 16
NW = NC * NS

OOV_PER_W = N_OOV // NW

WIN = 128
NWIN_FULL = VOCAB // WIN
TAIL_LO = NWIN_FULL * WIN
TAIL_W = VOCAB - TAIL_LO

CHW = 256
N_CHUNKS = 122
NBUF = 4

NV_IDX = N_KNOWN // 16
POS_BITS = 14
POS_MASK = (1 << POS_BITS) - 1
LOC_CAP = N_KNOWN + 32


def _sc_kernel(tab_hbm, idx_hbm, oov_hbm, tail_hbm, out_hbm,
               idx_v, loc_pk, chunks_v, tail_v, oov_v,
               stage_v, pbuf, cbuf, cstage, sh_pk,
               io_sem, stream_sems, oov_sem, app_sem, row_sems):
    wid = lax.axis_index("s") * NC + lax.axis_index("c")
    iota16 = lax.broadcasted_iota(jnp.int32, (16,), 0)

    def prefix16(x):
        p = x
        for s in (1, 2, 4, 8):
            pbuf[pl.ds(0, 16)] = p
            g = plsc.load_gather(pbuf, [jnp.maximum(iota16 - s, 0)])
            p = p + jnp.where(iota16 >= s, g, 0)
        return p

    def rank_of(pc):
        pbuf[pl.ds(0, 16)] = pc
        lo = iota16 * 0
        r1 = iota16 + 1
        for s in (8, 4, 2, 1):
            t = lo + s - 1
            v = plsc.load_gather(pbuf, [jnp.minimum(t, 15)])
            lo = lo + jnp.where((t <= 15) & (v < r1), s, 0)
        return jnp.minimum(lo, 15)

    idx_cp = pltpu.async_copy(idx_hbm, idx_v, io_sem)
    tail_cp = pltpu.async_copy(tail_hbm, tail_v, io_sem)
    oov_lo = wid * OOV_PER_W
    oov_in = pltpu.async_copy(
        oov_hbm.at[pl.ds(oov_lo, OOV_PER_W)], oov_v, oov_sem)

    lo_col = pl.multiple_of(
        jnp.where(wid < 4, 245 * WIN * wid, 244 * WIN * wid + 4 * WIN), WIN)
    n_win = jnp.where(wid < 4, 245, 244)
    hi_col = jnp.where(wid == 31, VOCAB, lo_col + n_win * WIN)

    def fire(c):
        col = pl.multiple_of(lo_col + c * CHW, WIN)
        slot = lax.rem(c, NBUF)
        return pltpu.async_copy(
            tab_hbm.at[:, pl.ds(col, CHW)],
            chunks_v.at[slot], stream_sems.at[slot])

    for b in range(NBUF):
        fire(jnp.int32(b))

    idx_cp.wait()
    neg1 = iota16 * 0 - 1

    def flush(acc, off, ap):
        @pl.when(ap > 0)
        def _():
            pltpu.make_async_copy(
                cstage.at[pl.ds(0, 16)],
                sh_pk.at[wid, pl.ds(0, 16)], app_sem).wait()

        cstage[pl.ds(0, 16)] = acc
        pltpu.async_copy(
            cstage.at[pl.ds(0, 16)],
            sh_pk.at[wid, pl.ds(pl.multiple_of(off, 16), 16)], app_sem)

    def bucket(k, carry):
        cnt, ap, acc = carry
        ids16 = idx_v[pl.ds(k * 16, 16)]
        m = (ids16 >= lo_col) & (ids16 < hi_col)
        nm = plsc.all_reduce_population_count(m)[0]

        def heavy(args):
            cnt, ap, acc = args
            pk16 = ((ids16 - lo_col) << POS_BITS) + (k * 16 + iota16)
            f = lax.rem(cnt, 16)
            pc = prefix16(m.astype(jnp.int32))
            sl = rank_of(pc)
            cbuf[pl.ds(0, 16)] = pk16
            pull = plsc.load_gather(cbuf, [sl])
            cbuf[pl.ds(0, 16)] = pull
            sh = jnp.maximum(iota16 - f, 0)
            new_lane = (iota16 >= f) & (iota16 < f + nm)
            merged = jnp.where(new_lane, plsc.load_gather(cbuf, [sh]), acc)
            full = f + nm >= 16

            @pl.when(full)
            def _():
                flush(merged, cnt - f, ap)

            ov = iota16 + (16 - f)
            ovm = iota16 < (f + nm - 16)
            spill = jnp.where(
                ovm, plsc.load_gather(cbuf, [jnp.minimum(ov, 15)]), neg1)
            acc2 = jnp.where(full, spill, merged)
            ap2 = jnp.where(full, 1, ap)
            return cnt + nm, ap2, acc2

        return lax.cond(nm > 0, heavy, lambda a: a, (cnt, ap, acc))

    cnt, ap, acc = lax.fori_loop(
        0, NV_IDX, bucket, (jnp.int32(0), jnp.int32(0), neg1))

    flush(acc, cnt - lax.rem(cnt, 16), ap)
    pltpu.make_async_copy(
        cstage.at[pl.ds(0, 16)], sh_pk.at[wid, pl.ds(0, 16)],
        app_sem).wait()
    pltpu.sync_copy(sh_pk.at[wid], loc_pk)

    nv = (cnt + 15) // 16

    oov_in.wait()
    oov_out = pltpu.async_copy(
        oov_v, out_hbm.at[pl.ds(N_KNOWN + oov_lo, OOV_PER_W)], oov_sem)

    def lane_drain(j):
        pltpu.make_async_copy(
            stage_v.at[pl.ds(j, 1)], out_hbm.at[pl.ds(0, 1)],
            row_sems.at[j]).wait()

    def extract(buf, bufsel, rel_lo, width, o):
        def vloop(v, o):
            pk16 = loc_pk[pl.ds(v * 16, 16)]
            rel16 = pk16 >> POS_BITS
            m = ((v * 16 + iota16) < cnt) & (rel16 >= rel_lo) & \
                (rel16 < rel_lo + width)
            nm = plsc.all_reduce_population_count(m)[0]
            mi = m.astype(jnp.int32)

            @pl.when(nm > 0)
            def _():
                for j in range(16):
                    @pl.when(mi[j] > 0)
                    def _():
                        @pl.when(o[j] > 0)
                        def _():
                            lane_drain(j)

                        colv = iota16 * 0 + (rel16[j] - rel_lo)
                        for q in range(4):
                            if bufsel is None:
                                vals = plsc.load_gather(
                                    buf, [iota16 + q * 16, colv])
                            else:
                                vals = plsc.load_gather(
                                    buf, [iota16 * 0 + bufsel,
                                          iota16 + q * 16, colv])
                            stage_v[j, pl.ds(q * 16, 16)] = vals
                        pltpu.async_copy(
                            stage_v.at[pl.ds(j, 1)],
                            out_hbm.at[pl.ds(pk16[j] & POS_MASK, 1)],
                            row_sems.at[j])

            return jnp.where(m, 1, o)

        return lax.fori_loop(0, nv, vloop, o)

    def chunk_loop(c, o):
        cur = lax.rem(c, NBUF)
        pltpu.make_async_copy(
            tab_hbm.at[:, pl.ds(0, CHW)], chunks_v.at[cur],
            stream_sems.at[cur]).wait()

        o = extract(chunks_v, cur, c * CHW, CHW, o)

        @pl.when(c + NBUF < N_CHUNKS)
        def _():
            fire(c + NBUF)

        return o

    o = lax.fori_loop(0, N_CHUNKS, chunk_loop, iota16 * 0)

    @pl.when(wid < 4)
    def _():
        col = pl.multiple_of(lo_col + N_CHUNKS * CHW, WIN)
        pltpu.async_copy(
            tab_hbm.at[:, pl.ds(col, WIN)],
            chunks_v.at[0, :, pl.ds(0, WIN)], stream_sems.at[0]).wait()

    o = lax.cond(
        wid < 4,
        lambda o: extract(chunks_v, jnp.int32(0), N_CHUNKS * CHW, WIN, o),
        lambda o: o, o)

    tail_cp.wait()
    o = lax.cond(
        wid == 31,
        lambda o: extract(tail_v, None, TAIL_LO - lo_col, TAIL_W, o),
        lambda o: o, o)

    for j in range(16):
        @pl.when(o[j] > 0)
        def _():
            lane_drain(j)

    oov_out.wait()


@jax.jit
def _run(tab_t, idx, oov, tail):
    k = functools.partial(
        pl.kernel,
        out_type=jax.ShapeDtypeStruct((N_KNOWN + N_OOV, EMBED_DIM), jnp.float32),
        mesh=plsc.VectorSubcoreMesh(core_axis_name="c", subcore_axis_name="s"),
        compiler_params=pltpu.CompilerParams(needs_layout_passes=False),
        scratch_types=[
            pltpu.VMEM((N_KNOWN,), jnp.int32),
            pltpu.VMEM((LOC_CAP,), jnp.int32),
            pltpu.VMEM((NBUF, EMBED_DIM, CHW), jnp.float32),
            pltpu.VMEM((EMBED_DIM, TAIL_W), jnp.float32),
            pltpu.VMEM((OOV_PER_W, EMBED_DIM), jnp.float32),
            pltpu.VMEM((16, EMBED_DIM), jnp.float32),
            pltpu.VMEM((16,), jnp.int32),
            pltpu.VMEM((16,), jnp.int32),
            pltpu.VMEM((16,), jnp.int32),
            pltpu.HBM((NW, LOC_CAP), jnp.int32),
            pltpu.SemaphoreType.DMA,
            pltpu.SemaphoreType.DMA((NBUF,)),
            pltpu.SemaphoreType.DMA,
            pltpu.SemaphoreType.DMA,
            pltpu.SemaphoreType.DMA((16,)),
        ],
    )(_sc_kernel)
    return k(tab_t, idx, oov, tail)


def kernel(embedding_table, prototype_token_ids, oov_embeddings):
    idx = prototype_token_ids.astype(jnp.int32)
    tab_t = embedding_table.T
    tail = tab_t[:, TAIL_LO:]
    return _run(tab_t, idx, oov_embeddings, tail)


# speedup vs baseline: 1.2375x; 1.2375x over previous
import functools

import jax
import jax.numpy as jnp
from jax import lax
from jax.experimental import pallas as pl
from jax.experimental.pallas import tpu as pltpu
from jax.experimental.pallas import tpu_sc as plsc

VOCAB = 1000000
EMBED_DIM = 64
N_KNOWN = 16384
N_OOV = 4096

NC = 2
NS = 16
NW = NC * NS

KNOWN_PER_W = N_KNOWN // NW
OOV_PER_W = N_OOV // NW
GS = 32
NG = KNOWN_PER_W // GS


def _sc_kernel(table_hbm, idx_hbm, oov_hbm, out_hbm,
               idx_v, rows_v, oov_v, gat_sem):
    wid = lax.axis_index("s") * NC + lax.axis_index("c")

    pltpu.sync_copy(idx_hbm.at[wid], idx_v)

    oov_base = wid * OOV_PER_W
    pltpu.sync_copy(oov_hbm.at[pl.ds(oov_base, OOV_PER_W)], oov_v)
    pltpu.sync_copy(oov_v, out_hbm.at[pl.ds(N_KNOWN + oov_base, OOV_PER_W)])

    def group(g, carry):
        base = g * GS
        cps = []
        for v in range(GS // 16):
            ids16 = idx_v[pl.ds(base + v * 16, 16)]
            for j in range(16):
                cps.append(pltpu.async_copy(
                    table_hbm.at[pl.ds(ids16[j], 1)],
                    rows_v.at[pl.ds(base + v * 16 + j, 1)],
                    gat_sem))
        for cp in cps:
            cp.wait()
        return carry

    lax.fori_loop(0, NG, group, 0)

    pltpu.sync_copy(rows_v, out_hbm.at[pl.ds(wid * KNOWN_PER_W, KNOWN_PER_W)])


@jax.jit
def _run(table, idx2d, oov):
    k = functools.partial(
        pl.kernel,
        out_type=jax.ShapeDtypeStruct((N_KNOWN + N_OOV, EMBED_DIM), jnp.float32),
        mesh=plsc.VectorSubcoreMesh(core_axis_name="c", subcore_axis_name="s"),
        scratch_types=[
            pltpu.VMEM((KNOWN_PER_W,), jnp.int32),
            pltpu.VMEM((KNOWN_PER_W, EMBED_DIM), jnp.float32),
            pltpu.VMEM((OOV_PER_W, EMBED_DIM), jnp.float32),
            pltpu.SemaphoreType.DMA,
        ],
    )(_sc_kernel)
    return k(table, idx2d, oov)


def kernel(embedding_table, prototype_token_ids, oov_embeddings):
    idx2d = prototype_token_ids.astype(jnp.int32).reshape(NW, KNOWN_PER_W)
    return _run(embedding_table, idx2d, oov_embeddings)
